# baseline (speedup 1.0000x reference)
import functools

import jax
import jax.numpy as jnp
from jax import lax
from jax.experimental import pallas as pl
from jax.experimental.pallas import tpu as pltpu
from jax.experimental.pallas import tpu_sc as plsc

H = 32
EH = 64
STEPS = 3
N_NODES = 10000
E_EDGES = 160000
NW = 32
CHUNK = 128
NCHUNK = 40
E_PAD = NW * NCHUNK * CHUNK

_MSG_TILE = 2048


def _msg_body(ef_ref, hs_ref, We1_ref, be1_ref, We2_ref, be2_ref, R_ref,
              out_ref):
    t = jnp.maximum(
        jnp.dot(ef_ref[...], We1_ref[...], preferred_element_type=jnp.float32)
        + be1_ref[...],
        0.0,
    )
    ew = (
        jnp.dot(t.astype(jnp.bfloat16), We2_ref[...],
                preferred_element_type=jnp.float32)
        + be2_ref[...]
    )
    rep = jnp.dot(hs_ref[...][:, :H].astype(jnp.bfloat16), R_ref[...],
                  preferred_element_type=jnp.float32)
    x = ew * rep
    x = x[:, :512] + x[:, 512:]
    x = x[:, :256] + x[:, 256:]
    x = x[:, :128] + x[:, 128:]
    x = x[:, :64] + x[:, 64:]
    x = x[:, :32] + x[:, 32:]
    row = (pl.program_id(0) * _MSG_TILE
           + lax.broadcasted_iota(jnp.int32, (_MSG_TILE, 1), 0))
    out_ref[...] = jnp.where(row < E_EDGES, x, 0.0)


def _msg_pallas(edge_feats, hs, We1, be1, We2b, be2, R):
    E = edge_feats.shape[0]
    D_BOND = edge_feats.shape[1]
    grid = E // _MSG_TILE
    return pl.pallas_call(
        _msg_body,
        grid=(grid,),
        in_specs=[
            pl.BlockSpec((_MSG_TILE, D_BOND), lambda i: (i, 0)),
            pl.BlockSpec((_MSG_TILE, 128), lambda i: (i, 0)),
            pl.BlockSpec((D_BOND, EH), lambda i: (0, 0)),
            pl.BlockSpec((1, EH), lambda i: (0, 0)),
            pl.BlockSpec((EH, H * H), lambda i: (0, 0)),
            pl.BlockSpec((1, H * H), lambda i: (0, 0)),
            pl.BlockSpec((H, H * H), lambda i: (0, 0)),
        ],
        out_specs=pl.BlockSpec((_MSG_TILE, H), lambda i: (i, 0)),
        out_shape=jax.ShapeDtypeStruct((E, H), jnp.float32),
    )(edge_feats, hs, We1, be1.reshape(1, EH), We2b, be2.reshape(1, H * H), R)


def _cat_body(ef_ref, hs_ref, Wpe_ref, bpe_ref, out_ref):
    emb = jnp.maximum(
        jnp.dot(ef_ref[...], Wpe_ref[...], preferred_element_type=jnp.float32)
        + bpe_ref[...],
        0.0,
    )
    row = (pl.program_id(0) * _MSG_TILE
           + lax.broadcasted_iota(jnp.int32, (_MSG_TILE, 1), 0))
    cat = jnp.concatenate([hs_ref[...][:, :H], emb], axis=1)
    out_ref[...] = jnp.where(row < E_EDGES, cat, 0.0)


def _cat_pallas(ef_pad, hs_pad, Wpe, bpe):
    D_BOND = ef_pad.shape[1]
    EO = Wpe.shape[1]
    grid = E_PAD // _MSG_TILE
    return pl.pallas_call(
        _cat_body,
        grid=(grid,),
        in_specs=[
            pl.BlockSpec((_MSG_TILE, D_BOND), lambda i: (i, 0)),
            pl.BlockSpec((_MSG_TILE, 128), lambda i: (i, 0)),
            pl.BlockSpec((D_BOND, EO), lambda i: (0, 0)),
            pl.BlockSpec((1, EO), lambda i: (0, 0)),
        ],
        out_specs=pl.BlockSpec((_MSG_TILE, H + EO), lambda i: (i, 0)),
        out_shape=jax.ShapeDtypeStruct((E_PAD, H + EO), jnp.float32),
    )(ef_pad, hs_pad, Wpe, bpe.reshape(1, EO))


_GRU_TILE = 2000


def _gru_body(agg_ref, h_ref, Wih_ref, bih_ref, Whh_ref, bhh_ref,
              out_ref):
    h = h_ref[...]
    agg = agg_ref[...]
    m = jnp.maximum(agg, 0.0) + h
    gi = jnp.dot(m, Wih_ref[...], preferred_element_type=jnp.float32) \
        + bih_ref[...]
    gh = jnp.dot(h, Whh_ref[...], preferred_element_type=jnp.float32) \
        + bhh_ref[...]
    r = jax.nn.sigmoid(gi[:, :H] + gh[:, :H])
    z = jax.nn.sigmoid(gi[:, H:2 * H] + gh[:, H:2 * H])
    n = jnp.tanh(gi[:, 2 * H:] + r * gh[:, 2 * H:])
    out_ref[...] = (1.0 - z) * n + z * h


def _gru_pallas(agg, h, W_ih, b_ih, W_hh, b_hh):
    N = h.shape[0]
    grid = N // _GRU_TILE
    WihT = W_ih.T
    WhhT = W_hh.T
    return pl.pallas_call(
        _gru_body,
        grid=(grid,),
        in_specs=[
            pl.BlockSpec((_GRU_TILE, H), lambda i: (i, 0)),
            pl.BlockSpec((_GRU_TILE, H), lambda i: (i, 0)),
            pl.BlockSpec((H, 3 * H), lambda i: (0, 0)),
            pl.BlockSpec((1, 3 * H), lambda i: (0, 0)),
            pl.BlockSpec((H, 3 * H), lambda i: (0, 0)),
            pl.BlockSpec((1, 3 * H), lambda i: (0, 0)),
        ],
        out_specs=pl.BlockSpec((_GRU_TILE, H), lambda i: (i, 0)),
        out_shape=jax.ShapeDtypeStruct((N, H), jnp.float32),
    )(agg, h, WihT, b_ih.reshape(1, 3 * H), WhhT, b_hh.reshape(1, 3 * H))


def _sc_gather_body(h_hbm, src_hbm, out_hbm, idx_v, rows_v, sem):
    c = lax.axis_index("c")
    s = lax.axis_index("s")
    w = c * 16 + s
    base = w * (NCHUNK * CHUNK)
    pltpu.sync_copy(src_hbm.at[w], idx_v)

    @pl.loop(0, NCHUNK)
    def _(j):
        pltpu.async_copy(h_hbm.at[idx_v.at[j]], rows_v, sem).wait()
        pltpu.sync_copy(rows_v, out_hbm.at[pl.ds(base + j * CHUNK, CHUNK)])


def _sc_gather(h, src3):
    assert h.shape[1] == 128
    mesh = plsc.VectorSubcoreMesh(core_axis_name="c", subcore_axis_name="s",
                                  num_cores=2, num_subcores=16)
    f = pl.kernel(
        _sc_gather_body,
        out_type=jax.ShapeDtypeStruct((E_PAD, 128), jnp.float32),
        mesh=mesh,
        scratch_types=[
            pltpu.VMEM((NCHUNK, CHUNK), jnp.int32),
            pltpu.VMEM((CHUNK, 128), jnp.float32),
            pltpu.SemaphoreType.DMA,
        ],
    )
    return f(h, src3)


def kernel(node_feats, edge_feats, edge_index, node_graph_ids, Wp, bp, We1,
           be1, We2, be2, W_ih, b_ih, W_hh, b_hh, Wpe, bpe, W1, b1, W2, b2,
           Wo, bo):
    N = node_feats.shape[0]
    G = 256
    NT = Wo.shape[1]
    src = edge_index[0]
    dst = edge_index[1]

    We2b = We2.astype(jnp.bfloat16)
    R = (jnp.arange(H * H)[None, :] // H == jnp.arange(H)[:, None]).astype(
        jnp.bfloat16)

    pad = E_PAD - E_EDGES
    ef_pad = jnp.pad(edge_feats, ((0, pad), (0, 0)))
    src_pad = jnp.pad(src, (0, pad))
    dst3 = jnp.pad(dst, (0, pad)).reshape(NW, NCHUNK, CHUNK)
    n_pad = 10240

    src3 = src_pad.reshape(NW, NCHUNK, CHUNK)
    dst_pad = dst3.reshape(-1)

    h = jax.nn.relu(node_feats @ Wp + bp)
    for _ in range(STEPS):
        h_pad = jnp.pad(h, ((0, n_pad - N), (0, 128 - H)))
        hs = _sc_gather(h_pad, src3)
        msg = _msg_pallas(ef_pad, hs, We1, be1, We2b, be2, R)
        agg = jax.ops.segment_sum(msg, dst_pad, num_segments=N)
        h = _gru_pallas(agg, h, W_ih, b_ih, W_hh, b_hh)

    h_pad = jnp.pad(h, ((0, n_pad - N), (0, 128 - H)))
    hs = _sc_gather(h_pad, src3)
    cat = _cat_pallas(ef_pad, hs, Wpe, bpe)
    gid_pad = jnp.pad(jnp.take(node_graph_ids, dst, axis=0), (0, pad))
    mol = jax.ops.segment_sum(cat, gid_pad, num_segments=G)
    mol = jax.nn.softmax(mol, axis=1)
    x1 = jax.nn.relu(mol @ W1 + b1)
    emb = jax.nn.relu(x1 @ W2 + b2)
    out = emb @ Wo + bo
    logits = out.reshape(-1, NT)
    return jax.nn.sigmoid(logits)


# speedup vs baseline: 1.0126x; 1.0126x over previous
import functools

import jax
import jax.numpy as jnp
from jax import lax
from jax.experimental import pallas as pl
from jax.experimental.pallas import tpu as pltpu
from jax.experimental.pallas import tpu_sc as plsc

H = 32
EH = 64
STEPS = 3
N_NODES = 10000
E_EDGES = 160000
NW = 32
CHUNK = 128
NCHUNK = 40
E_PAD = NW * NCHUNK * CHUNK

_MSG_TILE = 4096


def _msg_body(ef_ref, hs_ref, We1_ref, be1_ref, We2_ref, be2_ref, R_ref,
              out_ref):
    t = jnp.maximum(
        jnp.dot(ef_ref[...], We1_ref[...], preferred_element_type=jnp.float32)
        + be1_ref[...],
        0.0,
    )
    ew = (
        jnp.dot(t.astype(jnp.bfloat16), We2_ref[...],
                preferred_element_type=jnp.float32)
        + be2_ref[...]
    )
    rep = jnp.dot(hs_ref[...][:, :H].astype(jnp.bfloat16), R_ref[...],
                  preferred_element_type=jnp.float32)
    x = ew * rep
    x = x[:, :512] + x[:, 512:]
    x = x[:, :256] + x[:, 256:]
    x = x[:, :128] + x[:, 128:]
    x = x[:, :64] + x[:, 64:]
    x = x[:, :32] + x[:, 32:]
    row = (pl.program_id(0) * _MSG_TILE
           + lax.broadcasted_iota(jnp.int32, (_MSG_TILE, 1), 0))
    out_ref[...] = jnp.where(row < E_EDGES, x, 0.0)


def _msg_pallas(edge_feats, hs, We1, be1, We2b, be2, R):
    E = edge_feats.shape[0]
    D_BOND = edge_feats.shape[1]
    grid = E // _MSG_TILE
    return pl.pallas_call(
        _msg_body,
        grid=(grid,),
        in_specs=[
            pl.BlockSpec((_MSG_TILE, D_BOND), lambda i: (i, 0)),
            pl.BlockSpec((_MSG_TILE, 128), lambda i: (i, 0)),
            pl.BlockSpec((D_BOND, EH), lambda i: (0, 0)),
            pl.BlockSpec((1, EH), lambda i: (0, 0)),
            pl.BlockSpec((EH, H * H), lambda i: (0, 0)),
            pl.BlockSpec((1, H * H), lambda i: (0, 0)),
            pl.BlockSpec((H, H * H), lambda i: (0, 0)),
        ],
        out_specs=pl.BlockSpec((_MSG_TILE, H), lambda i: (i, 0)),
        out_shape=jax.ShapeDtypeStruct((E, H), jnp.float32),
    )(edge_feats, hs, We1, be1.reshape(1, EH), We2b, be2.reshape(1, H * H), R)


def _cat_body(ef_ref, hs_ref, Wpe_ref, bpe_ref, out_ref):
    emb = jnp.maximum(
        jnp.dot(ef_ref[...], Wpe_ref[...], preferred_element_type=jnp.float32)
        + bpe_ref[...],
        0.0,
    )
    row = (pl.program_id(0) * _MSG_TILE
           + lax.broadcasted_iota(jnp.int32, (_MSG_TILE, 1), 0))
    cat = jnp.concatenate([hs_ref[...][:, :H], emb], axis=1)
    out_ref[...] = jnp.where(row < E_EDGES, cat, 0.0)


def _cat_pallas(ef_pad, hs_pad, Wpe, bpe):
    D_BOND = ef_pad.shape[1]
    EO = Wpe.shape[1]
    grid = E_PAD // _MSG_TILE
    return pl.pallas_call(
        _cat_body,
        grid=(grid,),
        in_specs=[
            pl.BlockSpec((_MSG_TILE, D_BOND), lambda i: (i, 0)),
            pl.BlockSpec((_MSG_TILE, 128), lambda i: (i, 0)),
            pl.BlockSpec((D_BOND, EO), lambda i: (0, 0)),
            pl.BlockSpec((1, EO), lambda i: (0, 0)),
        ],
        out_specs=pl.BlockSpec((_MSG_TILE, H + EO), lambda i: (i, 0)),
        out_shape=jax.ShapeDtypeStruct((E_PAD, H + EO), jnp.float32),
    )(ef_pad, hs_pad, Wpe, bpe.reshape(1, EO))


_GRU_TILE = 2000


def _gru_body(agg_ref, h_ref, Wih_ref, bih_ref, Whh_ref, bhh_ref,
              out_ref):
    h = h_ref[...]
    agg = agg_ref[...]
    m = jnp.maximum(agg, 0.0) + h
    gi = jnp.dot(m, Wih_ref[...], preferred_element_type=jnp.float32) \
        + bih_ref[...]
    gh = jnp.dot(h, Whh_ref[...], preferred_element_type=jnp.float32) \
        + bhh_ref[...]
    r = jax.nn.sigmoid(gi[:, :H] + gh[:, :H])
    z = jax.nn.sigmoid(gi[:, H:2 * H] + gh[:, H:2 * H])
    n = jnp.tanh(gi[:, 2 * H:] + r * gh[:, 2 * H:])
    out_ref[...] = (1.0 - z) * n + z * h


def _gru_pallas(agg, h, W_ih, b_ih, W_hh, b_hh):
    N = h.shape[0]
    grid = N // _GRU_TILE
    WihT = W_ih.T
    WhhT = W_hh.T
    return pl.pallas_call(
        _gru_body,
        grid=(grid,),
        in_specs=[
            pl.BlockSpec((_GRU_TILE, H), lambda i: (i, 0)),
            pl.BlockSpec((_GRU_TILE, H), lambda i: (i, 0)),
            pl.BlockSpec((H, 3 * H), lambda i: (0, 0)),
            pl.BlockSpec((1, 3 * H), lambda i: (0, 0)),
            pl.BlockSpec((H, 3 * H), lambda i: (0, 0)),
            pl.BlockSpec((1, 3 * H), lambda i: (0, 0)),
        ],
        out_specs=pl.BlockSpec((_GRU_TILE, H), lambda i: (i, 0)),
        out_shape=jax.ShapeDtypeStruct((N, H), jnp.float32),
    )(agg, h, WihT, b_ih.reshape(1, 3 * H), WhhT, b_hh.reshape(1, 3 * H))


def _sc_gather_body(h_hbm, src_hbm, out_hbm, idx_v, rows_a, rows_b, sem_a,
                    sem_b):
    c = lax.axis_index("c")
    s = lax.axis_index("s")
    w = c * 16 + s
    base = w * (NCHUNK * CHUNK)
    pltpu.sync_copy(src_hbm.at[w], idx_v)
    pltpu.async_copy(h_hbm.at[idx_v.at[0]], rows_a, sem_a)

    @pl.loop(0, NCHUNK, step=2)
    def _(j):
        pltpu.async_copy(h_hbm.at[idx_v.at[j + 1]], rows_b, sem_b)
        pltpu.make_async_copy(h_hbm.at[pl.ds(0, CHUNK)], rows_a, sem_a).wait()
        pltpu.sync_copy(rows_a, out_hbm.at[pl.ds(base + j * CHUNK, CHUNK)])

        @pl.when(j + 2 < NCHUNK)
        def _():
            pltpu.async_copy(h_hbm.at[idx_v.at[j + 2]], rows_a, sem_a)

        pltpu.make_async_copy(h_hbm.at[pl.ds(0, CHUNK)], rows_b, sem_b).wait()
        pltpu.sync_copy(rows_b,
                        out_hbm.at[pl.ds(base + (j + 1) * CHUNK, CHUNK)])


def _sc_gather(h, src3):
    assert h.shape[1] == 128
    mesh = plsc.VectorSubcoreMesh(core_axis_name="c", subcore_axis_name="s",
                                  num_cores=2, num_subcores=16)
    f = pl.kernel(
        _sc_gather_body,
        out_type=jax.ShapeDtypeStruct((E_PAD, 128), jnp.float32),
        mesh=mesh,
        scratch_types=[
            pltpu.VMEM((NCHUNK, CHUNK), jnp.int32),
            pltpu.VMEM((CHUNK, 128), jnp.float32),
            pltpu.VMEM((CHUNK, 128), jnp.float32),
            pltpu.SemaphoreType.DMA,
            pltpu.SemaphoreType.DMA,
        ],
    )
    return f(h, src3)


def kernel(node_feats, edge_feats, edge_index, node_graph_ids, Wp, bp, We1,
           be1, We2, be2, W_ih, b_ih, W_hh, b_hh, Wpe, bpe, W1, b1, W2, b2,
           Wo, bo):
    N = node_feats.shape[0]
    G = 256
    NT = Wo.shape[1]
    src = edge_index[0]
    dst = edge_index[1]

    We2b = We2.astype(jnp.bfloat16)
    R = (jnp.arange(H * H)[None, :] // H == jnp.arange(H)[:, None]).astype(
        jnp.bfloat16)

    pad = E_PAD - E_EDGES
    ef_pad = jnp.pad(edge_feats, ((0, pad), (0, 0)))
    src_pad = jnp.pad(src, (0, pad))
    dst3 = jnp.pad(dst, (0, pad)).reshape(NW, NCHUNK, CHUNK)
    n_pad = 10240

    src3 = src_pad.reshape(NW, NCHUNK, CHUNK)
    dst_pad = dst3.reshape(-1)

    h = jax.nn.relu(node_feats @ Wp + bp)
    for _ in range(STEPS):
        h_pad = jnp.pad(h, ((0, n_pad - N), (0, 128 - H)))
        hs = _sc_gather(h_pad, src3)
        msg = _msg_pallas(ef_pad, hs, We1, be1, We2b, be2, R)
        agg = jax.ops.segment_sum(msg, dst_pad, num_segments=N)
        h = _gru_pallas(agg, h, W_ih, b_ih, W_hh, b_hh)

    h_pad = jnp.pad(h, ((0, n_pad - N), (0, 128 - H)))
    hs = _sc_gather(h_pad, src3)
    cat = _cat_pallas(ef_pad, hs, Wpe, bpe)
    gid_pad = jnp.pad(jnp.take(node_graph_ids, dst, axis=0), (0, pad))
    mol = jax.ops.segment_sum(cat, gid_pad, num_segments=G)
    mol = jax.nn.softmax(mol, axis=1)
    x1 = jax.nn.relu(mol @ W1 + b1)
    emb = jax.nn.relu(x1 @ W2 + b2)
    out = emb @ Wo + bo
    logits = out.reshape(-1, NT)
    return jax.nn.sigmoid(logits)


# speedup vs baseline: 1.1706x; 1.1560x over previous
import functools

import jax
import jax.numpy as jnp
from jax import lax
from jax.experimental import pallas as pl
from jax.experimental.pallas import tpu as pltpu
from jax.experimental.pallas import tpu_sc as plsc

H = 32
EH = 64
STEPS = 3
N_NODES = 10000
E_EDGES = 160000
NW = 32
CHUNK = 128
NCHUNK = 40
E_PAD = NW * NCHUNK * CHUNK

_MSG_TILE = 4096


def _msg_body(limit, ef_ref, hs_ref, We1_ref, be1_ref, We2_ref, be2_ref,
              R_ref, out_ref):
    t = jnp.maximum(
        jnp.dot(ef_ref[...], We1_ref[...], preferred_element_type=jnp.float32)
        + be1_ref[...],
        0.0,
    )
    ew = (
        jnp.dot(t.astype(jnp.bfloat16), We2_ref[...],
                preferred_element_type=jnp.float32)
        + be2_ref[...]
    )
    rep = jnp.dot(hs_ref[...][:, :H].astype(jnp.bfloat16), R_ref[...],
                  preferred_element_type=jnp.float32)
    x = ew * rep
    x = x[:, :512] + x[:, 512:]
    x = x[:, :256] + x[:, 256:]
    x = x[:, :128] + x[:, 128:]
    x = x[:, :64] + x[:, 64:]
    x = x[:, :32] + x[:, 32:]
    row = (pl.program_id(0) * _MSG_TILE
           + lax.broadcasted_iota(jnp.int32, (_MSG_TILE, 1), 0))
    out_ref[...] = jnp.where(row < limit, x, 0.0)


def _msg_pallas(edge_feats, hs, We1, be1, We2b, be2, R, limit):
    E = edge_feats.shape[0]
    D_BOND = edge_feats.shape[1]
    grid = E // _MSG_TILE
    return pl.pallas_call(
        functools.partial(_msg_body, limit),
        grid=(grid,),
        in_specs=[
            pl.BlockSpec((_MSG_TILE, D_BOND), lambda i: (i, 0)),
            pl.BlockSpec((_MSG_TILE, 128), lambda i: (i, 0)),
            pl.BlockSpec((D_BOND, EH), lambda i: (0, 0)),
            pl.BlockSpec((1, EH), lambda i: (0, 0)),
            pl.BlockSpec((EH, H * H), lambda i: (0, 0)),
            pl.BlockSpec((1, H * H), lambda i: (0, 0)),
            pl.BlockSpec((H, H * H), lambda i: (0, 0)),
        ],
        out_specs=pl.BlockSpec((_MSG_TILE, H), lambda i: (i, 0)),
        out_shape=jax.ShapeDtypeStruct((E, H), jnp.float32),
    )(edge_feats, hs, We1, be1.reshape(1, EH), We2b, be2.reshape(1, H * H), R)


def _cat_body(limit, ef_ref, hs_ref, Wpe_ref, bpe_ref, out_ref):
    emb = jnp.maximum(
        jnp.dot(ef_ref[...], Wpe_ref[...], preferred_element_type=jnp.float32)
        + bpe_ref[...],
        0.0,
    )
    row = (pl.program_id(0) * _MSG_TILE
           + lax.broadcasted_iota(jnp.int32, (_MSG_TILE, 1), 0))
    cat = jnp.concatenate([hs_ref[...][:, :H], emb], axis=1)
    out_ref[...] = jnp.where(row < limit, cat, 0.0)


def _cat_pallas(ef_pad, hs_pad, Wpe, bpe, limit):
    D_BOND = ef_pad.shape[1]
    EO = Wpe.shape[1]
    E = ef_pad.shape[0]
    grid = E // _MSG_TILE
    return pl.pallas_call(
        functools.partial(_cat_body, limit),
        grid=(grid,),
        in_specs=[
            pl.BlockSpec((_MSG_TILE, D_BOND), lambda i: (i, 0)),
            pl.BlockSpec((_MSG_TILE, 128), lambda i: (i, 0)),
            pl.BlockSpec((D_BOND, EO), lambda i: (0, 0)),
            pl.BlockSpec((1, EO), lambda i: (0, 0)),
        ],
        out_specs=pl.BlockSpec((_MSG_TILE, H + EO), lambda i: (i, 0)),
        out_shape=jax.ShapeDtypeStruct((E, H + EO), jnp.float32),
    )(ef_pad, hs_pad, Wpe, bpe.reshape(1, EO))


_GRU_TILE = 2000


def _gru_body(agg_ref, h_ref, Wih_ref, bih_ref, Whh_ref, bhh_ref,
              out_ref):
    h = h_ref[...]
    agg = agg_ref[...]
    m = jnp.maximum(agg, 0.0) + h
    gi = jnp.dot(m, Wih_ref[...], preferred_element_type=jnp.float32) \
        + bih_ref[...]
    gh = jnp.dot(h, Whh_ref[...], preferred_element_type=jnp.float32) \
        + bhh_ref[...]
    r = jax.nn.sigmoid(gi[:, :H] + gh[:, :H])
    z = jax.nn.sigmoid(gi[:, H:2 * H] + gh[:, H:2 * H])
    n = jnp.tanh(gi[:, 2 * H:] + r * gh[:, 2 * H:])
    out_ref[...] = (1.0 - z) * n + z * h


def _gru_pallas(agg, h, W_ih, b_ih, W_hh, b_hh):
    N = h.shape[0]
    grid = N // _GRU_TILE
    WihT = W_ih.T
    WhhT = W_hh.T
    return pl.pallas_call(
        _gru_body,
        grid=(grid,),
        in_specs=[
            pl.BlockSpec((_GRU_TILE, H), lambda i: (i, 0)),
            pl.BlockSpec((_GRU_TILE, H), lambda i: (i, 0)),
            pl.BlockSpec((H, 3 * H), lambda i: (0, 0)),
            pl.BlockSpec((1, 3 * H), lambda i: (0, 0)),
            pl.BlockSpec((H, 3 * H), lambda i: (0, 0)),
            pl.BlockSpec((1, 3 * H), lambda i: (0, 0)),
        ],
        out_specs=pl.BlockSpec((_GRU_TILE, H), lambda i: (i, 0)),
        out_shape=jax.ShapeDtypeStruct((N, H), jnp.float32),
    )(agg, h, WihT, b_ih.reshape(1, 3 * H), WhhT, b_hh.reshape(1, 3 * H))


def _sc_gather_body(nchunk, h_hbm, src_hbm, out_hbm, idx_v, rows_a, rows_b,
                    sem_a, sem_b):
    c = lax.axis_index("c")
    s = lax.axis_index("s")
    w = c * 16 + s
    base = w * (nchunk * CHUNK)
    pltpu.sync_copy(src_hbm.at[w], idx_v)
    pltpu.async_copy(h_hbm.at[idx_v.at[0]], rows_a, sem_a)

    @pl.loop(0, nchunk, step=2)
    def _(j):
        pltpu.async_copy(h_hbm.at[idx_v.at[j + 1]], rows_b, sem_b)
        pltpu.make_async_copy(h_hbm.at[pl.ds(0, CHUNK)], rows_a, sem_a).wait()
        pltpu.sync_copy(rows_a, out_hbm.at[pl.ds(base + j * CHUNK, CHUNK)])

        @pl.when(j + 2 < nchunk)
        def _():
            pltpu.async_copy(h_hbm.at[idx_v.at[j + 2]], rows_a, sem_a)

        pltpu.make_async_copy(h_hbm.at[pl.ds(0, CHUNK)], rows_b, sem_b).wait()
        pltpu.sync_copy(rows_b,
                        out_hbm.at[pl.ds(base + (j + 1) * CHUNK, CHUNK)])


def _sc_gather(h, src3):
    assert h.shape[1] == 128
    nw, nchunk, chunk = src3.shape
    assert nw == NW and chunk == CHUNK and nchunk % 2 == 0
    n_edges = nw * nchunk * chunk
    mesh = plsc.VectorSubcoreMesh(core_axis_name="c", subcore_axis_name="s",
                                  num_cores=2, num_subcores=16)
    f = pl.kernel(
        functools.partial(_sc_gather_body, nchunk),
        out_type=jax.ShapeDtypeStruct((n_edges, 128), jnp.float32),
        mesh=mesh,
        scratch_types=[
            pltpu.VMEM((nchunk, CHUNK), jnp.int32),
            pltpu.VMEM((CHUNK, 128), jnp.float32),
            pltpu.VMEM((CHUNK, 128), jnp.float32),
            pltpu.SemaphoreType.DMA,
            pltpu.SemaphoreType.DMA,
        ],
    )
    return f(h, src3)


def kernel(node_feats, edge_feats, edge_index, node_graph_ids, Wp, bp, We1,
           be1, We2, be2, W_ih, b_ih, W_hh, b_hh, Wpe, bpe, W1, b1, W2, b2,
           Wo, bo):
    N = node_feats.shape[0]
    G = 256
    NT = Wo.shape[1]
    src = edge_index[0]
    dst = edge_index[1]

    We2b = We2.astype(jnp.bfloat16)
    R = (jnp.arange(H * H)[None, :] // H == jnp.arange(H)[:, None]).astype(
        jnp.bfloat16)

    pad = E_PAD - E_EDGES
    ef_pad = jnp.pad(edge_feats, ((0, pad), (0, 0)))
    src_pad = jnp.pad(src, (0, pad))
    dst3 = jnp.pad(dst, (0, pad)).reshape(NW, NCHUNK, CHUNK)
    n_pad = 10240

    EHALF = E_PAD // 2
    dst_pad = dst3.reshape(-1)
    srcA3 = src_pad[:EHALF].reshape(NW, NCHUNK // 2, CHUNK)
    srcB3 = src_pad[EHALF:].reshape(NW, NCHUNK // 2, CHUNK)
    dstA = dst_pad[:EHALF]
    dstB = dst_pad[EHALF:]
    efA = ef_pad[:EHALF]
    efB = ef_pad[EHALF:]
    limB = E_EDGES - EHALF

    h = jax.nn.relu(node_feats @ Wp + bp)
    for _ in range(STEPS):
        h_pad = jnp.pad(h, ((0, n_pad - N), (0, 128 - H)))
        hsA = _sc_gather(h_pad, srcA3)
        msgA = _msg_pallas(efA, hsA, We1, be1, We2b, be2, R, EHALF)
        hsB = _sc_gather(h_pad, srcB3)
        msgB = _msg_pallas(efB, hsB, We1, be1, We2b, be2, R, limB)
        aggA = jax.ops.segment_sum(msgA, dstA, num_segments=N)
        aggB = jax.ops.segment_sum(msgB, dstB, num_segments=N)
        h = _gru_pallas(aggA + aggB, h, W_ih, b_ih, W_hh, b_hh)

    h_pad = jnp.pad(h, ((0, n_pad - N), (0, 128 - H)))
    gid_pad = jnp.pad(jnp.take(node_graph_ids, dst, axis=0), (0, pad))
    catA = _cat_pallas(efA, _sc_gather(h_pad, srcA3), Wpe, bpe, EHALF)
    catB = _cat_pallas(efB, _sc_gather(h_pad, srcB3), Wpe, bpe, limB)
    mol = (jax.ops.segment_sum(catA, gid_pad[:EHALF], num_segments=G)
           + jax.ops.segment_sum(catB, gid_pad[EHALF:], num_segments=G))
    mol = jax.nn.softmax(mol, axis=1)
    x1 = jax.nn.relu(mol @ W1 + b1)
    emb = jax.nn.relu(x1 @ W2 + b2)
    out = emb @ Wo + bo
    logits = out.reshape(-1, NT)
    return jax.nn.sigmoid(logits)
